# baseline (speedup 1.0000x reference)
import functools
import jax
import jax.numpy as jnp
from jax import lax
from jax.experimental import pallas as pl
from jax.experimental.pallas import tpu as pltpu
from jax.experimental.pallas import tpu_sc as plsc

_N0, _N2, _N3, _N1 = 64, 24, 24, 768
_NW = 32
_MAJ_PER_W = _N0 // _NW


def _sc_body(idx_hbm, lw_hbm, feat_hbm, out_hbm, idx_v, lw_v, buf, gate_ref):
    cid = lax.axis_index("c")
    sid = lax.axis_index("s")
    wid = sid * 2 + cid

    pltpu.sync_copy(idx_hbm, idx_v)
    pltpu.sync_copy(lw_hbm, lw_v)
    idx16 = idx_v[...]
    w16 = plsc.load_gather(lw_v, [idx16])
    gate_ref[...] = 1.0 / (1.0 + jnp.exp(-w16))

    def do_chunk(m, r):
        pltpu.sync_copy(feat_hbm.at[m, r], buf)
        gate = gate_ref[...]

        def row_body(rr, _):
            for c in range(_N1 // 16):
                sl = (rr, pl.ds(c * 16, 16))
                buf[sl] = buf[sl] * gate
            return 0

        lax.fori_loop(0, _N2, row_body, 0)
        pltpu.sync_copy(buf, out_hbm.at[m, r])

    def maj_body(j, _):
        m = wid * _MAJ_PER_W + j

        def r_body(r, __):
            do_chunk(m, r)
            return 0

        lax.fori_loop(0, _N2, r_body, 0)
        return 0

    lax.fori_loop(0, _MAJ_PER_W, maj_body, 0)


def kernel(idx, feat, layerweight):
    feat_t = jnp.transpose(feat, (0, 2, 3, 1))
    idx_b = jnp.full((16,), idx, dtype=jnp.int32)
    lw_pad = jnp.pad(layerweight[0], (0, 32 - layerweight.shape[1]))
    mesh = plsc.VectorSubcoreMesh(core_axis_name="c", subcore_axis_name="s")
    sck = functools.partial(
        pl.kernel,
        mesh=mesh,
        out_type=jax.ShapeDtypeStruct((_N0, _N2, _N3, _N1), jnp.float32),
        scratch_types=[
            pltpu.VMEM((16,), jnp.int32),
            pltpu.VMEM((32,), jnp.float32),
            pltpu.VMEM((_N3, _N1), jnp.float32),
            pltpu.VMEM((16,), jnp.float32),
        ],
        compiler_params=pltpu.CompilerParams(needs_layout_passes=False),
    )(_sc_body)
    out_t = sck(idx_b, lw_pad, feat_t)
    return jnp.transpose(out_t, (0, 3, 1, 2))


# speedup vs baseline: 2.3296x; 2.3296x over previous
import jax
import jax.numpy as jnp
from jax.experimental import pallas as pl
from jax.experimental.pallas import tpu as pltpu


def _gate_scale_kernel(idx_ref, lw_ref, feat_ref, out_ref):
    w = lw_ref[0, idx_ref[0]]
    gate = 1.0 / (1.0 + jnp.exp(-w))
    out_ref[...] = feat_ref[...] * gate


def kernel(idx, feat, layerweight):
    n0, n1, n2, n3 = feat.shape
    feat_t = jnp.transpose(feat, (0, 2, 3, 1))
    b0 = 8 if n0 % 8 == 0 else 1
    block = (b0, n2, n3, n1)
    idx_arr = jnp.asarray(idx, dtype=jnp.int32).reshape((1,))
    out_t = pl.pallas_call(
        _gate_scale_kernel,
        grid=(n0 // b0,),
        in_specs=[
            pl.BlockSpec(memory_space=pltpu.SMEM),
            pl.BlockSpec(memory_space=pltpu.SMEM),
            pl.BlockSpec(block, lambda i: (i, 0, 0, 0)),
        ],
        out_specs=pl.BlockSpec(block, lambda i: (i, 0, 0, 0)),
        out_shape=jax.ShapeDtypeStruct((n0, n2, n3, n1), feat.dtype),
        compiler_params=pltpu.CompilerParams(
            dimension_semantics=("parallel",),
            vmem_limit_bytes=128 * 1024 * 1024,
        ),
    )(idx_arr, layerweight, feat_t)
    return jnp.transpose(out_t, (0, 3, 1, 2))
